# baseline (speedup 1.0000x reference)
import functools

import jax
import jax.numpy as jnp
from jax.experimental import pallas as pl
from jax.experimental.pallas import tpu as pltpu
from jax.experimental.pallas import tpu_sc as plsc


def _shift_down(y):
    rows = jax.lax.broadcasted_iota(jnp.int32, y.shape, 0)
    return jnp.where(rows == 0, 0.0, pltpu.roll(y, 1, 0))


def _shift_up(y):
    rows = jax.lax.broadcasted_iota(jnp.int32, y.shape, 0)
    return jnp.where(rows == y.shape[0] - 1, 0.0, pltpu.roll(y, y.shape[0] - 1, 0))


def _conv3(hd, h, hu, w_ref):
    y = jnp.dot(hd, w_ref[0], preferred_element_type=jnp.float32)
    y += jnp.dot(h, w_ref[1], preferred_element_type=jnp.float32)
    y += jnp.dot(hu, w_ref[2], preferred_element_type=jnp.float32)
    return y


def _layer_norm(h, g, b):
    m = jnp.mean(h, axis=-1, keepdims=True)
    v = jnp.mean((h - m) ** 2, axis=-1, keepdims=True)
    return (h - m) * jax.lax.rsqrt(v + 1e-5) * g + b


def _predictor(xd, xb, xu, w1, b1, g1, be1, w2, b2, g2, be2, wl, bl):
    h = _conv3(xd, xb, xu, w1) + b1[...]
    h = jnp.maximum(h, 0.0)
    h = _layer_norm(h, g1[...], be1[...])
    h = _conv3(_shift_down(h), h, _shift_up(h), w2) + b2[...]
    h = jnp.maximum(h, 0.0)
    h = _layer_norm(h, g2[...], be2[...])
    return jnp.dot(h, wl[...], preferred_element_type=jnp.float32) + bl[0]


def _tc_body(x_ref,
             p_w1, p_b1, p_g1, p_be1, p_w2, p_b2, p_g2, p_be2, p_wl, p_bl,
             e_w1, e_b1, e_g1, e_be1, e_w2, e_b2, e_g2, e_be2, e_wl, e_bl,
             ppred_ref, epred_ref):
    xb = x_ref[0]
    xd, xu = _shift_down(xb), _shift_up(xb)
    ppred_ref[0] = _predictor(xd, xb, xu, p_w1, p_b1, p_g1, p_be1,
                              p_w2, p_b2, p_g2, p_be2, p_wl, p_bl)
    epred_ref[0] = _predictor(xd, xb, xu, e_w1, e_b1, e_g1, e_be1,
                              e_w2, e_b2, e_g2, e_be2, e_wl, e_bl)


def _tc_predictors(x, params):
    B, T, D = x.shape
    pp, ep = params["pitch_pred"], params["energy_pred"]

    def full(a):
        return pl.BlockSpec(a.shape, lambda b: (0,) * a.ndim)

    consts = [pp["W1"], pp["b1"], pp["g1"], pp["be1"],
              pp["W2"], pp["b2"], pp["g2"], pp["be2"],
              pp["Wl"], pp["bl"],
              ep["W1"], ep["b1"], ep["g1"], ep["be1"],
              ep["W2"], ep["b2"], ep["g2"], ep["be2"],
              ep["Wl"], ep["bl"]]

    ppred, epred = pl.pallas_call(
        _tc_body,
        grid=(B,),
        in_specs=[pl.BlockSpec((1, T, D), lambda b: (b, 0, 0))]
        + [full(c) for c in consts],
        out_specs=[pl.BlockSpec((1, T, 1), lambda b: (b, 0, 0)),
                   pl.BlockSpec((1, T, 1), lambda b: (b, 0, 0))],
        out_shape=[jax.ShapeDtypeStruct((B, T, 1), jnp.float32),
                   jax.ShapeDtypeStruct((B, T, 1), jnp.float32)],
        compiler_params=pltpu.CompilerParams(
            dimension_semantics=("parallel",)),
    )(x, *consts)
    return ppred.reshape(B, T), epred.reshape(B, T)


def _tc_bucket_body(pt_ref, et_ref, pbins_ref, ebins_ref, pidx_ref, eidx_ref):
    pidx_ref[...] = jnp.sum((pbins_ref[...] < pt_ref[...]).astype(jnp.int32),
                            axis=1, keepdims=True)
    eidx_ref[...] = jnp.sum((ebins_ref[...] < et_ref[...]).astype(jnp.int32),
                            axis=1, keepdims=True)


def _tc_bucketize(pt, et, pbins_pad, ebins_pad):
    BT = pt.shape[0]
    blk = 4096
    pb = pbins_pad.reshape(1, -1)
    eb = ebins_pad.reshape(1, -1)
    pidx, eidx = pl.pallas_call(
        _tc_bucket_body,
        grid=(BT // blk,),
        in_specs=[pl.BlockSpec((blk, 1), lambda i: (i, 0)),
                  pl.BlockSpec((blk, 1), lambda i: (i, 0)),
                  pl.BlockSpec(pb.shape, lambda i: (0, 0)),
                  pl.BlockSpec(eb.shape, lambda i: (0, 0))],
        out_specs=[pl.BlockSpec((blk, 1), lambda i: (i, 0)),
                   pl.BlockSpec((blk, 1), lambda i: (i, 0))],
        out_shape=[jax.ShapeDtypeStruct((BT, 1), jnp.int32),
                   jax.ShapeDtypeStruct((BT, 1), jnp.int32)],
        compiler_params=pltpu.CompilerParams(
            dimension_semantics=("parallel",)),
    )(pt.reshape(BT, 1), et.reshape(BT, 1), pb, eb)
    return pidx, eidx


_LANES = 16
_CHUNK = 128


def _sc_body(x_hbm, pidx_hbm, eidx_hbm, pemb_hbm, eemb_hbm,
             out_hbm,
             pidx, eidx, xbuf, prow, erow, psem, esem):
    info = plsc.get_sparse_core_info()
    nc, ns = info.num_cores, info.num_subcores
    nw = nc * ns
    per_w = x_hbm.shape[0] // nw
    nchunk = per_w // _CHUNK

    wid = jax.lax.axis_index("s") * nc + jax.lax.axis_index("c")
    base = wid * per_w

    pltpu.sync_copy(pidx_hbm.at[pl.ds(wid * nchunk, nchunk)], pidx)
    pltpu.sync_copy(eidx_hbm.at[pl.ds(wid * nchunk, nchunk)], eidx)

    groups = x_hbm.shape[1] // _LANES

    def chunk_step(c, carry):
        roff = base + c * _CHUNK
        pltpu.sync_copy(x_hbm.at[pl.ds(roff, _CHUNK)], xbuf)
        pcopy = pltpu.async_copy(pemb_hbm.at[pidx.at[c]], prow, psem)
        ecopy = pltpu.async_copy(eemb_hbm.at[eidx.at[c]], erow, esem)
        pcopy.wait()
        ecopy.wait()

        def add_row(j, inner):
            for k in range(groups):
                s = pl.ds(k * _LANES, _LANES)
                xbuf[j, s] = xbuf[j, s] + prow[j, s] + erow[j, s]
            return inner

        jax.lax.fori_loop(0, _CHUNK, add_row, 0)
        pltpu.sync_copy(xbuf, out_hbm.at[pl.ds(roff, _CHUNK)])
        return carry

    jax.lax.fori_loop(0, nchunk, chunk_step, 0)


def _sc_embed_add(x2d, pidx2d, eidx2d, pemb, eemb):
    BT, D = x2d.shape
    mesh = plsc.VectorSubcoreMesh(core_axis_name="c", subcore_axis_name="s")
    info = plsc.get_sparse_core_info()
    per_w = BT // (info.num_cores * info.num_subcores)
    nchunk = per_w // _CHUNK

    kern = functools.partial(
        pl.kernel,
        mesh=mesh,
        out_type=jax.ShapeDtypeStruct((BT, D), jnp.float32),
        scratch_types=[
            pltpu.VMEM((nchunk, _CHUNK), jnp.int32),
            pltpu.VMEM((nchunk, _CHUNK), jnp.int32),
            pltpu.VMEM((_CHUNK, D), jnp.float32),
            pltpu.VMEM((_CHUNK, D), jnp.float32),
            pltpu.VMEM((_CHUNK, D), jnp.float32),
            pltpu.SemaphoreType.DMA,
            pltpu.SemaphoreType.DMA,
        ],
    )(_sc_body)
    return kern(x2d, pidx2d, eidx2d, pemb, eemb)


def kernel(x, pitch_target, energy_target, params):
    B, T, D = x.shape
    BT = B * T

    ppred, epred = _tc_predictors(x, params)

    sentinel = jnp.full((1,), 2.0, jnp.float32)
    pbins_pad = jnp.concatenate([params["pitch_bins"], sentinel])
    ebins_pad = jnp.concatenate([params["energy_bins"], sentinel])

    pidx, eidx = _tc_bucketize(pitch_target.reshape(BT),
                               energy_target.reshape(BT),
                               pbins_pad, ebins_pad)

    x_out = _sc_embed_add(
        x.reshape(BT, D),
        pidx.reshape(BT // _CHUNK, _CHUNK),
        eidx.reshape(BT // _CHUNK, _CHUNK),
        params["pitch_embed"], params["energy_embed"])

    return (x_out.reshape(B, T, D), ppred, epred)
